# baseline (speedup 1.0000x reference)
import functools

import jax
import jax.numpy as jnp
from jax import lax
from jax.experimental import pallas as pl
from jax.experimental.pallas import tpu as pltpu
from jax.experimental.pallas import tpu_sc as plsc

N = 10000
E = 320000
G = 64
H = 256
C = 32
NSUB = 16
K = 128

_EDGES_PER_SUB = E // NSUB
_CHUNKS = 160
IB = 32
_L1_CHUNKS = 80
_L1_IB = 16
N_PAD = 10240
_ROWS_PER_SUB = N_PAD // NSUB
RB = 400
NB = N // RB
PREC = lax.Precision.DEFAULT


def _sc_agg_impl(table, src4, dst4, zeros_half, nblocks, ib, dst_by_worker):
    half = table.shape[1]
    mesh = plsc.VectorSubcoreMesh(core_axis_name="c", subcore_axis_name="s")

    @functools.partial(
        pl.kernel,
        out_type=jax.ShapeDtypeStruct((2, N_PAD, half), jnp.float32),
        mesh=mesh,
        scratch_types=[
            pltpu.VMEM((ib, K), jnp.int32),
            pltpu.VMEM((ib, K), jnp.int32),
            pltpu.VMEM((K, half), jnp.float32),
            pltpu.VMEM((K, half), jnp.float32),
            pltpu.VMEM_SHARED((N_PAD, half), jnp.float32),
            pltpu.SemaphoreType.DMA,
            pltpu.SemaphoreType.DMA,
            pltpu.SemaphoreType.DMA,
            pltpu.SemaphoreType.DMA,
        ],
    )
    def agg_kernel(h_hbm, src_hbm, dst_hbm, z_hbm, out_hbm,
                   src_v, dst_v, b0, b1, acc,
                   g0, g1, s0, s1):
        c = lax.axis_index("c")
        s = lax.axis_index("s")
        w = c * NSUB + s
        dw = w if dst_by_worker else s
        bufs = (b0, b1)
        gsems = (g0, g1)
        ssems = (s0, s1)
        nbuf = 2
        pltpu.sync_copy(z_hbm.at[pl.ds(s * _ROWS_PER_SUB, _ROWS_PER_SUB)],
                        acc.at[pl.ds(s * _ROWS_PER_SUB, _ROWS_PER_SUB)])
        plsc.subcore_barrier()

        @pl.loop(0, nblocks)
        def _(j):
            pltpu.sync_copy(src_hbm.at[w, j], src_v)
            pltpu.sync_copy(dst_hbm.at[dw, j], dst_v)
            for b in range(nbuf):
                pltpu.async_copy(h_hbm.at[src_v.at[b]], bufs[b], gsems[b])

            @pl.loop(0, ib // nbuf - 1)
            def _(p):
                i = nbuf * p
                for b in range(nbuf):
                    pltpu.make_async_copy(
                        h_hbm.at[src_v.at[i + b]], bufs[b], gsems[b]).wait()
                    pltpu.async_copy(
                        bufs[b], acc.at[dst_v.at[i + b]], ssems[b], add=True)
                for b in range(nbuf):
                    pltpu.make_async_copy(
                        bufs[b], acc.at[dst_v.at[i + b]], ssems[b]).wait()
                    pltpu.async_copy(
                        h_hbm.at[src_v.at[i + nbuf + b]], bufs[b], gsems[b])

            i = ib - nbuf
            for b in range(nbuf):
                pltpu.make_async_copy(
                    h_hbm.at[src_v.at[i + b]], bufs[b], gsems[b]).wait()
                pltpu.async_copy(
                    bufs[b], acc.at[dst_v.at[i + b]], ssems[b], add=True)
            for b in range(nbuf):
                pltpu.make_async_copy(
                    bufs[b], acc.at[dst_v.at[i + b]], ssems[b]).wait()

        plsc.subcore_barrier()
        pltpu.sync_copy(acc.at[pl.ds(s * _ROWS_PER_SUB, _ROWS_PER_SUB)],
                        out_hbm.at[c, pl.ds(s * _ROWS_PER_SUB, _ROWS_PER_SUB)])

    return agg_kernel(table, src4, dst4, zeros_half)


def _sc_agg(h_stacked, src4, dst4, zeros_half):
    return _sc_agg_impl(h_stacked, src4, dst4, zeros_half,
                        _CHUNKS // IB, IB, dst_by_worker=False)


def _sc_agg_edges(x, src4, dst4, zeros128):
    return _sc_agg_impl(x, src4, dst4, zeros128,
                        _L1_CHUNKS // _L1_IB, _L1_IB, dst_by_worker=True)


def _mlp_bn_body(halves, hA, hB, aA, aB, W1r, b1r, W2r, b2r, scr, gmr, btr,
                 hn_ref, y_scr, ssum, ssq):
    t = pl.program_id(0)

    @pl.when(t < NB)
    def _():
        e = scr[0, 0]
        if halves:
            z = jnp.concatenate([e * hA[0] + aA[0], e * hB[0] + aB[0]], axis=1)
        else:
            z = e * jnp.concatenate([hA[...], hB[...]], axis=1) + aA[0] + aB[0]
        tt = jnp.maximum(
            jnp.dot(z, W1r[...], precision=PREC, preferred_element_type=jnp.float32)
            + b1r[...], 0.0)
        y = jnp.maximum(
            jnp.dot(tt, W2r[...], precision=PREC, preferred_element_type=jnp.float32)
            + b2r[...], 0.0)
        y_scr[pl.ds(t * RB, RB), :] = y

        @pl.when(t == 0)
        def _():
            ssum[...] = jnp.zeros_like(ssum)
            ssq[...] = jnp.zeros_like(ssq)

        ssum[...] += jnp.sum(y, axis=0, keepdims=True)
        ssq[...] += jnp.sum(y * y, axis=0, keepdims=True)

    @pl.when(t >= NB)
    def _():
        i = t - NB
        y = y_scr[pl.ds(i * RB, RB), :]
        mu = ssum[...] * (1.0 / N)
        var = ssq[...] * (1.0 / N) - mu * mu
        a = gmr[...] / jnp.sqrt(var + 1e-5)
        bb = btr[...] - mu * a
        hn = y * a + bb
        hn_ref[...] = jnp.stack([hn[:, :H // 2], hn[:, H // 2:]], axis=0)


def _mlp_bn_layer(hin, hinB, agg3d, W1, b1, W2, b2, scal, gamma, beta, halves):
    fa = agg3d.shape[2]

    def rowix(t):
        return jnp.where(t < NB, t, NB - 1)

    if halves:
        h_specs = [
            pl.BlockSpec((1, RB, H // 2), lambda t: (0, rowix(t), 0)),
            pl.BlockSpec((1, RB, H // 2), lambda t: (1, rowix(t), 0)),
        ]
    else:
        fin = hin.shape[1]
        h_specs = [
            pl.BlockSpec((RB, fin), lambda t: (rowix(t), 0)),
            pl.BlockSpec((RB, fin), lambda t: (rowix(t) + NB, 0)),
        ]
    return pl.pallas_call(
        functools.partial(_mlp_bn_body, halves),
        grid=(2 * NB,),
        in_specs=h_specs + [
            pl.BlockSpec((1, RB, fa), lambda t: (0, rowix(t), 0)),
            pl.BlockSpec((1, RB, fa), lambda t: (1, rowix(t), 0)),
            pl.BlockSpec(W1.shape, lambda t: (0, 0)),
            pl.BlockSpec((1, H), lambda t: (0, 0)),
            pl.BlockSpec((H, H), lambda t: (0, 0)),
            pl.BlockSpec((1, H), lambda t: (0, 0)),
            pl.BlockSpec((1, 128), lambda t: (0, 0)),
            pl.BlockSpec((1, H), lambda t: (0, 0)),
            pl.BlockSpec((1, H), lambda t: (0, 0)),
        ],
        out_specs=pl.BlockSpec((2, RB, H // 2),
                               lambda t: (0, jnp.where(t >= NB, t - NB, 0), 0)),
        out_shape=jax.ShapeDtypeStruct((2, N, H // 2), jnp.float32),
        scratch_shapes=[
            pltpu.VMEM((N, H), jnp.float32),
            pltpu.VMEM((1, H), jnp.float32),
            pltpu.VMEM((1, H), jnp.float32),
        ],
    )(hin, hinB, agg3d, agg3d, W1, b1, W2, b2, scal, gamma, beta)


def _pool_body(h1a, h1b, h2a, h2b, h3a, h3b, h4a, h4b, bt,
               W1r, b1r, W2r, b2r, out, acc, cnt):
    i = pl.program_id(0)
    hb = jnp.concatenate(
        [r[0] for r in (h1a, h1b, h2a, h2b, h3a, h3b, h4a, h4b)], axis=1)
    ohT = (bt[0] == lax.broadcasted_iota(jnp.int32, (G, RB), 0)).astype(jnp.float32)

    @pl.when(i == 0)
    def _():
        acc[...] = jnp.zeros_like(acc)
        cnt[...] = jnp.zeros_like(cnt)

    acc[...] += lax.dot_general(ohT, hb, (((1,), (0,)), ((), ())),
                                precision=PREC, preferred_element_type=jnp.float32)
    cnt[...] += lax.dot_general(ohT, jnp.ones((RB, 1), jnp.float32),
                                (((1,), (0,)), ((), ())),
                                precision=PREC, preferred_element_type=jnp.float32)

    @pl.when(i == NB - 1)
    def _():
        pooled = acc[...] / jnp.maximum(cnt[...], 1.0)
        zz = jnp.maximum(
            jnp.dot(pooled, W1r[...], precision=PREC,
                    preferred_element_type=jnp.float32) + b1r[...], 0.0)
        lg = jnp.dot(zz, W2r[...], precision=PREC,
                     preferred_element_type=jnp.float32) + b2r[...]
        m = jnp.max(lg, axis=1, keepdims=True)
        out[...] = lg - m - jnp.log(jnp.sum(jnp.exp(lg - m), axis=1, keepdims=True))


def _pool_head(hs, batch3d, W1, b1, W2, b2):
    in_specs = []
    args = []
    for h in hs:
        args += [h, h]
        in_specs += [pl.BlockSpec((1, RB, H // 2), lambda i: (0, i, 0)),
                     pl.BlockSpec((1, RB, H // 2), lambda i: (1, i, 0))]
    args += [batch3d, W1, b1, W2, b2]
    in_specs += [
        pl.BlockSpec((1, 1, RB), lambda i: (i, 0, 0)),
        pl.BlockSpec((4 * H, H), lambda i: (0, 0)),
        pl.BlockSpec((1, H), lambda i: (0, 0)),
        pl.BlockSpec((H, C), lambda i: (0, 0)),
        pl.BlockSpec((1, C), lambda i: (0, 0)),
    ]
    return pl.pallas_call(
        _pool_body,
        grid=(NB,),
        in_specs=in_specs,
        out_specs=pl.BlockSpec((G, C), lambda i: (0, 0)),
        out_shape=jax.ShapeDtypeStruct((G, C), jnp.float32),
        scratch_shapes=[
            pltpu.VMEM((G, 4 * H), jnp.float32),
            pltpu.VMEM((G, 1), jnp.float32),
        ],
    )(*args)


def kernel(x, edge_index, batch, params, lin1_W, lin1_b, lin2_W, lin2_b):
    x = x.astype(jnp.float32)
    src = edge_index[0].astype(jnp.int32)
    dst = edge_index[1].astype(jnp.int32)
    nw = 2 * NSUB
    pad_l1 = _L1_CHUNKS * K - E // nw
    src_w1 = jnp.concatenate(
        [src.reshape(nw, E // nw), jnp.zeros((nw, pad_l1), jnp.int32)], axis=1)
    padrow1 = N + jnp.arange(nw, dtype=jnp.int32)[:, None]
    dst_w1 = jnp.concatenate(
        [dst.reshape(nw, E // nw),
         jnp.broadcast_to(padrow1, (nw, pad_l1))], axis=1)
    src4d_l1 = src_w1.reshape(nw, _L1_CHUNKS // _L1_IB, _L1_IB, K)
    dst4d_l1 = dst_w1.reshape(nw, _L1_CHUNKS // _L1_IB, _L1_IB, K)
    pad_h = _CHUNKS * K - _EDGES_PER_SUB
    src_w = jnp.concatenate(
        [src.reshape(NSUB, _EDGES_PER_SUB), jnp.zeros((NSUB, pad_h), jnp.int32)],
        axis=1)
    padrow = N + jnp.arange(NSUB, dtype=jnp.int32)[:, None]
    dst_w = jnp.concatenate(
        [dst.reshape(NSUB, _EDGES_PER_SUB),
         jnp.broadcast_to(padrow, (NSUB, pad_h))], axis=1)
    src4d = jnp.concatenate([src_w, src_w + N]).reshape(nw, _CHUNKS // IB, IB, K)
    dst4d = dst_w.reshape(NSUB, _CHUNKS // IB, IB, K)
    batch3d = batch.astype(jnp.int32).reshape(NB, 1, RB)
    z128 = jnp.zeros((N_PAD, H // 2), jnp.float32)

    fin_half = x.shape[1] // 2
    h_stacked = x.reshape(N, 2, fin_half).transpose(1, 0, 2).reshape(2 * N, fin_half)

    hs = []
    h3 = None
    for li, p in enumerate(params):
        if li == 0:
            agg = _sc_agg_edges(x, src4d_l1, dst4d_l1, z128)
            hin = hinB = h_stacked
        else:
            agg = _sc_agg(h3.reshape(2 * N, H // 2), src4d, dst4d, z128)
            hin = hinB = h3
        scal = jnp.broadcast_to(jnp.reshape(1.0 + p['eps'], (1, 1)), (1, 128))
        h3 = _mlp_bn_layer(hin, hinB, agg,
                           p['W1'], p['b1'].reshape(1, H),
                           p['W2'], p['b2'].reshape(1, H), scal,
                           p['gamma'].reshape(1, H), p['beta'].reshape(1, H),
                           halves=(li != 0))
        hs.append(h3)

    return _pool_head(hs, batch3d, lin1_W, lin1_b.reshape(1, H),
                      lin2_W, lin2_b.reshape(1, C))


# speedup vs baseline: 1.9231x; 1.9231x over previous
import functools

import jax
import jax.numpy as jnp
from jax import lax
from jax.experimental import pallas as pl
from jax.experimental.pallas import tpu as pltpu
from jax.experimental.pallas import tpu_sc as plsc

N = 10000
E = 320000
G = 64
H = 256
C = 32
NSUB = 16
K = 100

_EDGES_PER_SUB = E // NSUB
_CHUNKS = 200
IB = 50
_L1_CHUNKS = 100
_L1_IB = 50
N_PAD = 10240
_ROWS_PER_SUB = N_PAD // NSUB
RB = 400
NB = N // RB
PREC = lax.Precision.DEFAULT


def _sc_agg_impl(table, src4, dst4, zeros_half, nblocks, ib, dst_by_worker):
    half = table.shape[1]
    mesh = plsc.VectorSubcoreMesh(core_axis_name="c", subcore_axis_name="s")

    @functools.partial(
        pl.kernel,
        out_type=jax.ShapeDtypeStruct((2, N_PAD, half), jnp.float32),
        mesh=mesh,
        scratch_types=[
            pltpu.VMEM((ib, K), jnp.int32),
            pltpu.VMEM((ib, K), jnp.int32),
            pltpu.VMEM((K, half), jnp.float32),
            pltpu.VMEM((K, half), jnp.float32),
            pltpu.VMEM_SHARED((N_PAD, half), jnp.float32),
            pltpu.SemaphoreType.DMA,
            pltpu.SemaphoreType.DMA,
            pltpu.SemaphoreType.DMA,
            pltpu.SemaphoreType.DMA,
        ],
    )
    def agg_kernel(h_hbm, src_hbm, dst_hbm, z_hbm, out_hbm,
                   src_v, dst_v, b0, b1, acc,
                   g0, g1, s0, s1):
        c = lax.axis_index("c")
        s = lax.axis_index("s")
        w = c * NSUB + s
        dw = w if dst_by_worker else s
        bufs = (b0, b1)
        gsems = (g0, g1)
        ssems = (s0, s1)
        nbuf = 2
        pltpu.sync_copy(z_hbm.at[pl.ds(s * _ROWS_PER_SUB, _ROWS_PER_SUB)],
                        acc.at[pl.ds(s * _ROWS_PER_SUB, _ROWS_PER_SUB)])
        plsc.subcore_barrier()

        @pl.loop(0, nblocks)
        def _(j):
            pltpu.sync_copy(src_hbm.at[w, j], src_v)
            pltpu.sync_copy(dst_hbm.at[dw, j], dst_v)
            for b in range(nbuf):
                pltpu.async_copy(h_hbm.at[src_v.at[b]], bufs[b], gsems[b])

            @pl.loop(0, ib // nbuf - 1)
            def _(p):
                i = nbuf * p
                for b in range(nbuf):
                    pltpu.make_async_copy(
                        h_hbm.at[src_v.at[i + b]], bufs[b], gsems[b]).wait()
                    pltpu.async_copy(
                        bufs[b], acc.at[dst_v.at[i + b]], ssems[b], add=True)
                for b in range(nbuf):
                    pltpu.make_async_copy(
                        bufs[b], acc.at[dst_v.at[i + b]], ssems[b]).wait()
                    pltpu.async_copy(
                        h_hbm.at[src_v.at[i + nbuf + b]], bufs[b], gsems[b])

            i = ib - nbuf
            for b in range(nbuf):
                pltpu.make_async_copy(
                    h_hbm.at[src_v.at[i + b]], bufs[b], gsems[b]).wait()
                pltpu.async_copy(
                    bufs[b], acc.at[dst_v.at[i + b]], ssems[b], add=True)
            for b in range(nbuf):
                pltpu.make_async_copy(
                    bufs[b], acc.at[dst_v.at[i + b]], ssems[b]).wait()

        plsc.subcore_barrier()
        pltpu.sync_copy(acc.at[pl.ds(s * _ROWS_PER_SUB, _ROWS_PER_SUB)],
                        out_hbm.at[c, pl.ds(s * _ROWS_PER_SUB, _ROWS_PER_SUB)])

    return agg_kernel(table, src4, dst4, zeros_half)


def _sc_agg(h_stacked, src4, dst4, zeros_half):
    return _sc_agg_impl(h_stacked, src4, dst4, zeros_half,
                        _CHUNKS // IB, IB, dst_by_worker=False)


def _sc_agg_edges(x, src4, dst4, zeros128):
    return _sc_agg_impl(x, src4, dst4, zeros128,
                        _L1_CHUNKS // _L1_IB, _L1_IB, dst_by_worker=True)


def _mlp_bn_body(halves, hA, hB, aA, aB, W1r, b1r, W2r, b2r, scr, gmr, btr,
                 hn_ref, y_scr, ssum, ssq):
    t = pl.program_id(0)

    @pl.when(t < NB)
    def _():
        e = scr[0, 0]
        if halves:
            z = jnp.concatenate([e * hA[0] + aA[0], e * hB[0] + aB[0]], axis=1)
        else:
            z = e * jnp.concatenate([hA[...], hB[...]], axis=1) + aA[0] + aB[0]
        tt = jnp.maximum(
            jnp.dot(z, W1r[...], precision=PREC, preferred_element_type=jnp.float32)
            + b1r[...], 0.0)
        y = jnp.maximum(
            jnp.dot(tt, W2r[...], precision=PREC, preferred_element_type=jnp.float32)
            + b2r[...], 0.0)
        y_scr[pl.ds(t * RB, RB), :] = y

        @pl.when(t == 0)
        def _():
            ssum[...] = jnp.zeros_like(ssum)
            ssq[...] = jnp.zeros_like(ssq)

        ssum[...] += jnp.sum(y, axis=0, keepdims=True)
        ssq[...] += jnp.sum(y * y, axis=0, keepdims=True)

    @pl.when(t >= NB)
    def _():
        i = t - NB
        y = y_scr[pl.ds(i * RB, RB), :]
        mu = ssum[...] * (1.0 / N)
        var = ssq[...] * (1.0 / N) - mu * mu
        a = gmr[...] / jnp.sqrt(var + 1e-5)
        bb = btr[...] - mu * a
        hn = y * a + bb
        hn_ref[...] = jnp.stack([hn[:, :H // 2], hn[:, H // 2:]], axis=0)


def _mlp_bn_layer(hin, hinB, agg3d, W1, b1, W2, b2, scal, gamma, beta, halves):
    fa = agg3d.shape[2]

    def rowix(t):
        return jnp.where(t < NB, t, NB - 1)

    if halves:
        h_specs = [
            pl.BlockSpec((1, RB, H // 2), lambda t: (0, rowix(t), 0)),
            pl.BlockSpec((1, RB, H // 2), lambda t: (1, rowix(t), 0)),
        ]
    else:
        fin = hin.shape[1]
        h_specs = [
            pl.BlockSpec((RB, fin), lambda t: (rowix(t), 0)),
            pl.BlockSpec((RB, fin), lambda t: (rowix(t) + NB, 0)),
        ]
    return pl.pallas_call(
        functools.partial(_mlp_bn_body, halves),
        grid=(2 * NB,),
        in_specs=h_specs + [
            pl.BlockSpec((1, RB, fa), lambda t: (0, rowix(t), 0)),
            pl.BlockSpec((1, RB, fa), lambda t: (1, rowix(t), 0)),
            pl.BlockSpec(W1.shape, lambda t: (0, 0)),
            pl.BlockSpec((1, H), lambda t: (0, 0)),
            pl.BlockSpec((H, H), lambda t: (0, 0)),
            pl.BlockSpec((1, H), lambda t: (0, 0)),
            pl.BlockSpec((1, 128), lambda t: (0, 0)),
            pl.BlockSpec((1, H), lambda t: (0, 0)),
            pl.BlockSpec((1, H), lambda t: (0, 0)),
        ],
        out_specs=pl.BlockSpec((2, RB, H // 2),
                               lambda t: (0, jnp.where(t >= NB, t - NB, 0), 0)),
        out_shape=jax.ShapeDtypeStruct((2, N, H // 2), jnp.float32),
        scratch_shapes=[
            pltpu.VMEM((N, H), jnp.float32),
            pltpu.VMEM((1, H), jnp.float32),
            pltpu.VMEM((1, H), jnp.float32),
        ],
    )(hin, hinB, agg3d, agg3d, W1, b1, W2, b2, scal, gamma, beta)


def _pool_body(h1a, h1b, h2a, h2b, h3a, h3b, h4a, h4b, bt,
               W1r, b1r, W2r, b2r, out, acc, cnt):
    i = pl.program_id(0)
    hb = jnp.concatenate(
        [r[0] for r in (h1a, h1b, h2a, h2b, h3a, h3b, h4a, h4b)], axis=1)
    ohT = (bt[0] == lax.broadcasted_iota(jnp.int32, (G, RB), 0)).astype(jnp.float32)

    @pl.when(i == 0)
    def _():
        acc[...] = jnp.zeros_like(acc)
        cnt[...] = jnp.zeros_like(cnt)

    acc[...] += lax.dot_general(ohT, hb, (((1,), (0,)), ((), ())),
                                precision=PREC, preferred_element_type=jnp.float32)
    cnt[...] += lax.dot_general(ohT, jnp.ones((RB, 1), jnp.float32),
                                (((1,), (0,)), ((), ())),
                                precision=PREC, preferred_element_type=jnp.float32)

    @pl.when(i == NB - 1)
    def _():
        pooled = acc[...] / jnp.maximum(cnt[...], 1.0)
        zz = jnp.maximum(
            jnp.dot(pooled, W1r[...], precision=PREC,
                    preferred_element_type=jnp.float32) + b1r[...], 0.0)
        lg = jnp.dot(zz, W2r[...], precision=PREC,
                     preferred_element_type=jnp.float32) + b2r[...]
        m = jnp.max(lg, axis=1, keepdims=True)
        out[...] = lg - m - jnp.log(jnp.sum(jnp.exp(lg - m), axis=1, keepdims=True))


def _pool_head(hs, batch3d, W1, b1, W2, b2):
    in_specs = []
    args = []
    for h in hs:
        args += [h, h]
        in_specs += [pl.BlockSpec((1, RB, H // 2), lambda i: (0, i, 0)),
                     pl.BlockSpec((1, RB, H // 2), lambda i: (1, i, 0))]
    args += [batch3d, W1, b1, W2, b2]
    in_specs += [
        pl.BlockSpec((1, 1, RB), lambda i: (i, 0, 0)),
        pl.BlockSpec((4 * H, H), lambda i: (0, 0)),
        pl.BlockSpec((1, H), lambda i: (0, 0)),
        pl.BlockSpec((H, C), lambda i: (0, 0)),
        pl.BlockSpec((1, C), lambda i: (0, 0)),
    ]
    return pl.pallas_call(
        _pool_body,
        grid=(NB,),
        in_specs=in_specs,
        out_specs=pl.BlockSpec((G, C), lambda i: (0, 0)),
        out_shape=jax.ShapeDtypeStruct((G, C), jnp.float32),
        scratch_shapes=[
            pltpu.VMEM((G, 4 * H), jnp.float32),
            pltpu.VMEM((G, 1), jnp.float32),
        ],
    )(*args)


def kernel(x, edge_index, batch, params, lin1_W, lin1_b, lin2_W, lin2_b):
    x = x.astype(jnp.float32)
    src = edge_index[0].astype(jnp.int32)
    dst = edge_index[1].astype(jnp.int32)
    nw = 2 * NSUB
    pad_l1 = _L1_CHUNKS * K - E // nw
    src_w1 = jnp.concatenate(
        [src.reshape(nw, E // nw), jnp.zeros((nw, pad_l1), jnp.int32)], axis=1)
    padrow1 = N + jnp.arange(nw, dtype=jnp.int32)[:, None]
    dst_w1 = jnp.concatenate(
        [dst.reshape(nw, E // nw),
         jnp.broadcast_to(padrow1, (nw, pad_l1))], axis=1)
    src4d_l1 = src_w1.reshape(nw, _L1_CHUNKS // _L1_IB, _L1_IB, K)
    dst4d_l1 = dst_w1.reshape(nw, _L1_CHUNKS // _L1_IB, _L1_IB, K)
    pad_h = _CHUNKS * K - _EDGES_PER_SUB
    src_w = jnp.concatenate(
        [src.reshape(NSUB, _EDGES_PER_SUB), jnp.zeros((NSUB, pad_h), jnp.int32)],
        axis=1)
    padrow = N + jnp.arange(NSUB, dtype=jnp.int32)[:, None]
    dst_w = jnp.concatenate(
        [dst.reshape(NSUB, _EDGES_PER_SUB),
         jnp.broadcast_to(padrow, (NSUB, pad_h))], axis=1)
    src4d = jnp.concatenate([src_w, src_w + N]).reshape(nw, _CHUNKS // IB, IB, K)
    dst4d = dst_w.reshape(NSUB, _CHUNKS // IB, IB, K)
    batch3d = batch.astype(jnp.int32).reshape(NB, 1, RB)
    z128 = jnp.zeros((N_PAD, H // 2), jnp.float32)

    fin_half = x.shape[1] // 2
    h_stacked = x.reshape(N, 2, fin_half).transpose(1, 0, 2).reshape(2 * N, fin_half)

    hs = []
    h3 = None
    for li, p in enumerate(params):
        if li == 0:
            agg = _sc_agg_edges(x, src4d_l1, dst4d_l1, z128)
            hin = hinB = h_stacked
        else:
            agg = _sc_agg(h3.reshape(2 * N, H // 2), src4d, dst4d, z128)
            hin = hinB = h3
        scal = jnp.broadcast_to(jnp.reshape(1.0 + p['eps'], (1, 1)), (1, 128))
        h3 = _mlp_bn_layer(hin, hinB, agg,
                           p['W1'], p['b1'].reshape(1, H),
                           p['W2'], p['b2'].reshape(1, H), scal,
                           p['gamma'].reshape(1, H), p['beta'].reshape(1, H),
                           halves=(li != 0))
        hs.append(h3)

    return _pool_head(hs, batch3d, lin1_W, lin1_b.reshape(1, H),
                      lin2_W, lin2_b.reshape(1, C))


# speedup vs baseline: 1.9371x; 1.0073x over previous
import functools

import jax
import jax.numpy as jnp
from jax import lax
from jax.experimental import pallas as pl
from jax.experimental.pallas import tpu as pltpu
from jax.experimental.pallas import tpu_sc as plsc

N = 10000
E = 320000
G = 64
H = 256
C = 32
NSUB = 16
K = 100

_EDGES_PER_SUB = E // NSUB
_CHUNKS = 200
IB = 50
_L1_CHUNKS = 100
_L1_IB = 50
N_PAD = 10240
_ROWS_PER_SUB = N_PAD // NSUB
RB = 400
NB = N // RB
PREC = lax.Precision.DEFAULT


def _sc_agg_impl(table, src4, dst4, zeros_half, nblocks, ib, dst_by_worker):
    half = table.shape[1]
    mesh = plsc.VectorSubcoreMesh(core_axis_name="c", subcore_axis_name="s")

    @functools.partial(
        pl.kernel,
        out_type=jax.ShapeDtypeStruct((2, N_PAD, half), jnp.float32),
        mesh=mesh,
        scratch_types=[
            pltpu.VMEM((ib, K), jnp.int32),
            pltpu.VMEM((ib, K), jnp.int32),
            pltpu.VMEM((K, half), jnp.float32),
            pltpu.VMEM((K, half), jnp.float32),
            pltpu.VMEM_SHARED((N_PAD, half), jnp.float32),
            pltpu.SemaphoreType.DMA,
            pltpu.SemaphoreType.DMA,
            pltpu.SemaphoreType.DMA,
            pltpu.SemaphoreType.DMA,
        ],
    )
    def agg_kernel(h_hbm, src_hbm, dst_hbm, z_hbm, out_hbm,
                   src_v, dst_v, b0, b1, acc,
                   g0, g1, s0, s1):
        c = lax.axis_index("c")
        s = lax.axis_index("s")
        w = c * NSUB + s
        dw = w if dst_by_worker else s
        bufs = (b0, b1)
        gsems = (g0, g1)
        ssems = (s0, s1)
        nbuf = 2
        pltpu.sync_copy(src_hbm.at[w, 0], src_v)
        pltpu.sync_copy(dst_hbm.at[dw, 0], dst_v)
        for b in range(nbuf):
            pltpu.async_copy(h_hbm.at[src_v.at[b]], bufs[b], gsems[b])
        pltpu.sync_copy(z_hbm.at[pl.ds(s * _ROWS_PER_SUB, _ROWS_PER_SUB)],
                        acc.at[pl.ds(s * _ROWS_PER_SUB, _ROWS_PER_SUB)])
        plsc.subcore_barrier()

        @pl.loop(0, nblocks)
        def _(j):
            @pl.when(j > 0)
            def _():
                pltpu.sync_copy(src_hbm.at[w, j], src_v)
                pltpu.sync_copy(dst_hbm.at[dw, j], dst_v)
                for b in range(nbuf):
                    pltpu.async_copy(h_hbm.at[src_v.at[b]], bufs[b], gsems[b])

            @pl.loop(0, ib // nbuf - 1)
            def _(p):
                i = nbuf * p
                for b in range(nbuf):
                    pltpu.make_async_copy(
                        h_hbm.at[src_v.at[i + b]], bufs[b], gsems[b]).wait()
                    pltpu.async_copy(
                        bufs[b], acc.at[dst_v.at[i + b]], ssems[b], add=True)
                for b in range(nbuf):
                    pltpu.make_async_copy(
                        bufs[b], acc.at[dst_v.at[i + b]], ssems[b]).wait()
                    pltpu.async_copy(
                        h_hbm.at[src_v.at[i + nbuf + b]], bufs[b], gsems[b])

            i = ib - nbuf
            for b in range(nbuf):
                pltpu.make_async_copy(
                    h_hbm.at[src_v.at[i + b]], bufs[b], gsems[b]).wait()
                pltpu.async_copy(
                    bufs[b], acc.at[dst_v.at[i + b]], ssems[b], add=True)
            for b in range(nbuf):
                pltpu.make_async_copy(
                    bufs[b], acc.at[dst_v.at[i + b]], ssems[b]).wait()

        plsc.subcore_barrier()
        pltpu.sync_copy(acc.at[pl.ds(s * _ROWS_PER_SUB, _ROWS_PER_SUB)],
                        out_hbm.at[c, pl.ds(s * _ROWS_PER_SUB, _ROWS_PER_SUB)])

    return agg_kernel(table, src4, dst4, zeros_half)


def _sc_agg(h_stacked, src4, dst4, zeros_half):
    return _sc_agg_impl(h_stacked, src4, dst4, zeros_half,
                        _CHUNKS // IB, IB, dst_by_worker=False)


def _sc_agg_edges(x, src4, dst4, zeros128):
    return _sc_agg_impl(x, src4, dst4, zeros128,
                        _L1_CHUNKS // _L1_IB, _L1_IB, dst_by_worker=True)


def _mlp_bn_body(halves, hA, hB, aA, aB, W1r, b1r, W2r, b2r, scr, gmr, btr,
                 hn_ref, y_scr, ssum, ssq):
    t = pl.program_id(0)

    @pl.when(t < NB)
    def _():
        e = scr[0, 0]
        if halves:
            z = jnp.concatenate([e * hA[0] + aA[0], e * hB[0] + aB[0]], axis=1)
        else:
            z = e * jnp.concatenate([hA[...], hB[...]], axis=1) + aA[0] + aB[0]
        tt = jnp.maximum(
            jnp.dot(z, W1r[...], precision=PREC, preferred_element_type=jnp.float32)
            + b1r[...], 0.0)
        y = jnp.maximum(
            jnp.dot(tt, W2r[...], precision=PREC, preferred_element_type=jnp.float32)
            + b2r[...], 0.0)
        y_scr[pl.ds(t * RB, RB), :] = y

        @pl.when(t == 0)
        def _():
            ssum[...] = jnp.zeros_like(ssum)
            ssq[...] = jnp.zeros_like(ssq)

        ssum[...] += jnp.sum(y, axis=0, keepdims=True)
        ssq[...] += jnp.sum(y * y, axis=0, keepdims=True)

    @pl.when(t >= NB)
    def _():
        i = t - NB
        y = y_scr[pl.ds(i * RB, RB), :]
        mu = ssum[...] * (1.0 / N)
        var = ssq[...] * (1.0 / N) - mu * mu
        a = gmr[...] / jnp.sqrt(var + 1e-5)
        bb = btr[...] - mu * a
        hn = y * a + bb
        hn_ref[...] = jnp.stack([hn[:, :H // 2], hn[:, H // 2:]], axis=0)


def _mlp_bn_layer(hin, hinB, agg3d, W1, b1, W2, b2, scal, gamma, beta, halves):
    fa = agg3d.shape[2]

    def rowix(t):
        return jnp.where(t < NB, t, NB - 1)

    if halves:
        h_specs = [
            pl.BlockSpec((1, RB, H // 2), lambda t: (0, rowix(t), 0)),
            pl.BlockSpec((1, RB, H // 2), lambda t: (1, rowix(t), 0)),
        ]
    else:
        fin = hin.shape[1]
        h_specs = [
            pl.BlockSpec((RB, fin), lambda t: (rowix(t), 0)),
            pl.BlockSpec((RB, fin), lambda t: (rowix(t) + NB, 0)),
        ]
    return pl.pallas_call(
        functools.partial(_mlp_bn_body, halves),
        grid=(2 * NB,),
        in_specs=h_specs + [
            pl.BlockSpec((1, RB, fa), lambda t: (0, rowix(t), 0)),
            pl.BlockSpec((1, RB, fa), lambda t: (1, rowix(t), 0)),
            pl.BlockSpec(W1.shape, lambda t: (0, 0)),
            pl.BlockSpec((1, H), lambda t: (0, 0)),
            pl.BlockSpec((H, H), lambda t: (0, 0)),
            pl.BlockSpec((1, H), lambda t: (0, 0)),
            pl.BlockSpec((1, 128), lambda t: (0, 0)),
            pl.BlockSpec((1, H), lambda t: (0, 0)),
            pl.BlockSpec((1, H), lambda t: (0, 0)),
        ],
        out_specs=pl.BlockSpec((2, RB, H // 2),
                               lambda t: (0, jnp.where(t >= NB, t - NB, 0), 0)),
        out_shape=jax.ShapeDtypeStruct((2, N, H // 2), jnp.float32),
        scratch_shapes=[
            pltpu.VMEM((N, H), jnp.float32),
            pltpu.VMEM((1, H), jnp.float32),
            pltpu.VMEM((1, H), jnp.float32),
        ],
    )(hin, hinB, agg3d, agg3d, W1, b1, W2, b2, scal, gamma, beta)


def _pool_body(h1a, h1b, h2a, h2b, h3a, h3b, h4a, h4b, bt,
               W1r, b1r, W2r, b2r, out, acc, cnt):
    i = pl.program_id(0)
    hb = jnp.concatenate(
        [r[0] for r in (h1a, h1b, h2a, h2b, h3a, h3b, h4a, h4b)], axis=1)
    ohT = (bt[0] == lax.broadcasted_iota(jnp.int32, (G, RB), 0)).astype(jnp.float32)

    @pl.when(i == 0)
    def _():
        acc[...] = jnp.zeros_like(acc)
        cnt[...] = jnp.zeros_like(cnt)

    acc[...] += lax.dot_general(ohT, hb, (((1,), (0,)), ((), ())),
                                precision=PREC, preferred_element_type=jnp.float32)
    cnt[...] += lax.dot_general(ohT, jnp.ones((RB, 1), jnp.float32),
                                (((1,), (0,)), ((), ())),
                                precision=PREC, preferred_element_type=jnp.float32)

    @pl.when(i == NB - 1)
    def _():
        pooled = acc[...] / jnp.maximum(cnt[...], 1.0)
        zz = jnp.maximum(
            jnp.dot(pooled, W1r[...], precision=PREC,
                    preferred_element_type=jnp.float32) + b1r[...], 0.0)
        lg = jnp.dot(zz, W2r[...], precision=PREC,
                     preferred_element_type=jnp.float32) + b2r[...]
        m = jnp.max(lg, axis=1, keepdims=True)
        out[...] = lg - m - jnp.log(jnp.sum(jnp.exp(lg - m), axis=1, keepdims=True))


def _pool_head(hs, batch3d, W1, b1, W2, b2):
    in_specs = []
    args = []
    for h in hs:
        args += [h, h]
        in_specs += [pl.BlockSpec((1, RB, H // 2), lambda i: (0, i, 0)),
                     pl.BlockSpec((1, RB, H // 2), lambda i: (1, i, 0))]
    args += [batch3d, W1, b1, W2, b2]
    in_specs += [
        pl.BlockSpec((1, 1, RB), lambda i: (i, 0, 0)),
        pl.BlockSpec((4 * H, H), lambda i: (0, 0)),
        pl.BlockSpec((1, H), lambda i: (0, 0)),
        pl.BlockSpec((H, C), lambda i: (0, 0)),
        pl.BlockSpec((1, C), lambda i: (0, 0)),
    ]
    return pl.pallas_call(
        _pool_body,
        grid=(NB,),
        in_specs=in_specs,
        out_specs=pl.BlockSpec((G, C), lambda i: (0, 0)),
        out_shape=jax.ShapeDtypeStruct((G, C), jnp.float32),
        scratch_shapes=[
            pltpu.VMEM((G, 4 * H), jnp.float32),
            pltpu.VMEM((G, 1), jnp.float32),
        ],
    )(*args)


def kernel(x, edge_index, batch, params, lin1_W, lin1_b, lin2_W, lin2_b):
    x = x.astype(jnp.float32)
    src = edge_index[0].astype(jnp.int32)
    dst = edge_index[1].astype(jnp.int32)
    nw = 2 * NSUB
    pad_l1 = _L1_CHUNKS * K - E // nw
    src_w1 = jnp.concatenate(
        [src.reshape(nw, E // nw), jnp.zeros((nw, pad_l1), jnp.int32)], axis=1)
    padrow1 = N + jnp.arange(nw, dtype=jnp.int32)[:, None]
    dst_w1 = jnp.concatenate(
        [dst.reshape(nw, E // nw),
         jnp.broadcast_to(padrow1, (nw, pad_l1))], axis=1)
    src4d_l1 = src_w1.reshape(nw, _L1_CHUNKS // _L1_IB, _L1_IB, K)
    dst4d_l1 = dst_w1.reshape(nw, _L1_CHUNKS // _L1_IB, _L1_IB, K)
    pad_h = _CHUNKS * K - _EDGES_PER_SUB
    src_w = jnp.concatenate(
        [src.reshape(NSUB, _EDGES_PER_SUB), jnp.zeros((NSUB, pad_h), jnp.int32)],
        axis=1)
    padrow = N + jnp.arange(NSUB, dtype=jnp.int32)[:, None]
    dst_w = jnp.concatenate(
        [dst.reshape(NSUB, _EDGES_PER_SUB),
         jnp.broadcast_to(padrow, (NSUB, pad_h))], axis=1)
    src4d = jnp.concatenate([src_w, src_w + N]).reshape(nw, _CHUNKS // IB, IB, K)
    dst4d = dst_w.reshape(NSUB, _CHUNKS // IB, IB, K)
    batch3d = batch.astype(jnp.int32).reshape(NB, 1, RB)
    z128 = jnp.zeros((N_PAD, H // 2), jnp.float32)

    fin_half = x.shape[1] // 2
    h_stacked = x.reshape(N, 2, fin_half).transpose(1, 0, 2).reshape(2 * N, fin_half)

    hs = []
    h3 = None
    for li, p in enumerate(params):
        if li == 0:
            agg = _sc_agg_edges(x, src4d_l1, dst4d_l1, z128)
            hin = hinB = h_stacked
        else:
            agg = _sc_agg(h3.reshape(2 * N, H // 2), src4d, dst4d, z128)
            hin = hinB = h3
        scal = jnp.broadcast_to(jnp.reshape(1.0 + p['eps'], (1, 1)), (1, 128))
        h3 = _mlp_bn_layer(hin, hinB, agg,
                           p['W1'], p['b1'].reshape(1, H),
                           p['W2'], p['b2'].reshape(1, H), scal,
                           p['gamma'].reshape(1, H), p['beta'].reshape(1, H),
                           halves=(li != 0))
        hs.append(h3)

    return _pool_head(hs, batch3d, lin1_W, lin1_b.reshape(1, H),
                      lin2_W, lin2_b.reshape(1, C))
